# baseline (speedup 1.0000x reference)
import functools

import jax
import jax.numpy as jnp
from jax import lax
from jax.experimental import pallas as pl
from jax.experimental.pallas import tpu as pltpu
from jax.experimental.pallas import tpu_sc as plsc

LANES = 16


def _build_sc_kernel(B, K, b_per_w, chunk, num_cores):
    mesh = plsc.VectorSubcoreMesh(core_axis_name="c", subcore_axis_name="s")
    n_chunks = b_per_w // chunk
    K2 = 2 * K

    @functools.partial(
        pl.kernel,
        out_type=jax.ShapeDtypeStruct((B,), jnp.float32),
        mesh=mesh,
        compiler_params=pltpu.CompilerParams(needs_layout_passes=False),
        scratch_types=[
            pltpu.VMEM((b_per_w,), jnp.int32),
            pltpu.VMEM((b_per_w,), jnp.int32),
            pltpu.VMEM((b_per_w,), jnp.int32),
            pltpu.VMEM((b_per_w,), jnp.int32),
            pltpu.VMEM((b_per_w,), jnp.int32),
            pltpu.VMEM((b_per_w,), jnp.int32),
            pltpu.VMEM((chunk, K2), jnp.float32),
            pltpu.VMEM((chunk, K2), jnp.float32),
            pltpu.VMEM((chunk, K2), jnp.float32),
            pltpu.VMEM((b_per_w,), jnp.float32),
            pltpu.SemaphoreType.DMA,
            pltpu.SemaphoreType.DMA,
            pltpu.SemaphoreType.DMA,
        ],
    )
    def sc_kernel(pidx0_hbm, pidx1_hbm, pidx2_hbm, col0_hbm, col1_hbm,
                  col2_hbm, f0_hbm, f1_hbm, f2_hbm, out_hbm,
                  pidx0_v, pidx1_v, pidx2_v, col0_v, col1_v, col2_v,
                  r0_v, r1_v, r2_v, out_v, sem0, sem1, sem2):
        wid = lax.axis_index("s") * num_cores + lax.axis_index("c")
        base = wid * b_per_w

        pltpu.sync_copy(pidx0_hbm.at[pl.ds(base, b_per_w)], pidx0_v)
        pltpu.sync_copy(pidx1_hbm.at[pl.ds(base, b_per_w)], pidx1_v)
        pltpu.sync_copy(pidx2_hbm.at[pl.ds(base, b_per_w)], pidx2_v)
        pltpu.sync_copy(col0_hbm.at[pl.ds(base, b_per_w)], col0_v)
        pltpu.sync_copy(col1_hbm.at[pl.ds(base, b_per_w)], col1_v)
        pltpu.sync_copy(col2_hbm.at[pl.ds(base, b_per_w)], col2_v)

        lane = lax.iota(jnp.int32, LANES)

        def do_chunk(c, carry):
            off = c * chunk
            cp0 = pltpu.async_copy(f0_hbm.at[pidx0_v.at[pl.ds(off, chunk)]],
                                   r0_v, sem0)
            cp1 = pltpu.async_copy(f1_hbm.at[pidx1_v.at[pl.ds(off, chunk)]],
                                   r1_v, sem1)
            cp2 = pltpu.async_copy(f2_hbm.at[pidx2_v.at[pl.ds(off, chunk)]],
                                   r2_v, sem2)
            cp0.wait()
            cp1.wait()
            cp2.wait()

            def do_group(g, carry2):
                bvec = g * LANES + lane
                col0 = col0_v[pl.ds(off + g * LANES, LANES)]
                col1 = col1_v[pl.ds(off + g * LANES, LANES)]
                col2 = col2_v[pl.ds(off + g * LANES, LANES)]
                acc = jnp.zeros((LANES,), jnp.float32)
                for k in range(K):
                    v0 = plsc.load_gather(r0_v, [bvec, col0 + k])
                    v1 = plsc.load_gather(r1_v, [bvec, col1 + k])
                    v2 = plsc.load_gather(r2_v, [bvec, col2 + k])
                    acc = acc + v0 * v1 * v2
                out_v[pl.ds(off + g * LANES, LANES)] = acc
                return carry2

            lax.fori_loop(0, chunk // LANES, do_group, 0)
            return carry

        lax.fori_loop(0, n_chunks, do_chunk, 0)

        pltpu.sync_copy(out_v, out_hbm.at[pl.ds(base, b_per_w)])

    return sc_kernel


def kernel(indices, F0, F1, F2):
    B = indices.shape[0]
    V, K = F0.shape
    info = plsc.get_sparse_core_info()
    num_workers = info.num_cores * info.num_subcores
    b_per_w = B // num_workers
    chunk = min(b_per_w, 256)
    F0p = lax.optimization_barrier(F0.reshape(V // 2, 2 * K))
    F1p = lax.optimization_barrier(F1.reshape(V // 2, 2 * K))
    F2p = lax.optimization_barrier(F2.reshape(V // 2, 2 * K))
    pidx = indices >> 1
    col = (indices & 1) * K
    sc = _build_sc_kernel(B, K, b_per_w, chunk, info.num_cores)
    return sc(pidx[:, 0], pidx[:, 1], pidx[:, 2],
              col[:, 0], col[:, 1], col[:, 2],
              F0p, F1p, F2p)


# speedup vs baseline: 1.6784x; 1.6784x over previous
import functools

import jax
import jax.numpy as jnp
from jax import lax
from jax.experimental import pallas as pl
from jax.experimental.pallas import tpu as pltpu
from jax.experimental.pallas import tpu_sc as plsc

LANES = 16


def _build_sc_kernel(B, K, b_per_w, chunk, num_cores):
    mesh = plsc.VectorSubcoreMesh(core_axis_name="c", subcore_axis_name="s")
    n_chunks = b_per_w // chunk

    @functools.partial(
        pl.kernel,
        out_type=jax.ShapeDtypeStruct((B,), jnp.float32),
        mesh=mesh,
        compiler_params=pltpu.CompilerParams(needs_layout_passes=False),
        scratch_types=[
            pltpu.VMEM((b_per_w,), jnp.int32),
            pltpu.VMEM((b_per_w,), jnp.int32),
            pltpu.VMEM((b_per_w,), jnp.int32),
            pltpu.VMEM((chunk, K), jnp.float32),
            pltpu.VMEM((chunk, K), jnp.float32),
            pltpu.VMEM((chunk, K), jnp.float32),
            pltpu.VMEM((b_per_w,), jnp.float32),
            pltpu.SemaphoreType.DMA,
            pltpu.SemaphoreType.DMA,
            pltpu.SemaphoreType.DMA,
        ],
    )
    def sc_kernel(idx0_hbm, idx1_hbm, idx2_hbm, f0_hbm, f1_hbm, f2_hbm,
                  out_hbm, idx0_v, idx1_v, idx2_v, r0_v, r1_v, r2_v, out_v,
                  sem0, sem1, sem2):
        wid = lax.axis_index("s") * num_cores + lax.axis_index("c")
        base = wid * b_per_w

        pltpu.sync_copy(idx0_hbm.at[pl.ds(base, b_per_w)], idx0_v)
        pltpu.sync_copy(idx1_hbm.at[pl.ds(base, b_per_w)], idx1_v)
        pltpu.sync_copy(idx2_hbm.at[pl.ds(base, b_per_w)], idx2_v)

        lane = lax.iota(jnp.int32, LANES)
        perms = [jnp.bitwise_xor(lane, s) for s in (8, 4, 2, 1)]

        def do_chunk(c, carry):
            off = c * chunk

            def fire(g, carry2):
                iv0 = idx0_v[pl.ds(off + g * LANES, LANES)]
                iv1 = idx1_v[pl.ds(off + g * LANES, LANES)]
                iv2 = idx2_v[pl.ds(off + g * LANES, LANES)]
                for l in range(LANES):
                    b = g * LANES + l
                    pltpu.make_async_copy(
                        f0_hbm.at[pl.ds(iv0[l], 1), :],
                        r0_v.at[pl.ds(b, 1), :], sem0).start()
                    pltpu.make_async_copy(
                        f1_hbm.at[pl.ds(iv1[l], 1), :],
                        r1_v.at[pl.ds(b, 1), :], sem1).start()
                    pltpu.make_async_copy(
                        f2_hbm.at[pl.ds(iv2[l], 1), :],
                        r2_v.at[pl.ds(b, 1), :], sem2).start()
                return carry2

            lax.fori_loop(0, chunk // LANES, fire, 0)

            pltpu.make_async_copy(f0_hbm.at[pl.ds(0, chunk), :], r0_v,
                                  sem0).wait()
            pltpu.make_async_copy(f1_hbm.at[pl.ds(0, chunk), :], r1_v,
                                  sem1).wait()
            pltpu.make_async_copy(f2_hbm.at[pl.ds(0, chunk), :], r2_v,
                                  sem2).wait()

            def do_group(g, carry2):
                vec = jnp.zeros((LANES,), jnp.float32)
                for l in range(LANES):
                    b = g * LANES + l
                    acc = (r0_v[b, pl.ds(0, LANES)]
                           * r1_v[b, pl.ds(0, LANES)]
                           * r2_v[b, pl.ds(0, LANES)])
                    for j in range(1, K // LANES):
                        acc = acc + (r0_v[b, pl.ds(j * LANES, LANES)]
                                     * r1_v[b, pl.ds(j * LANES, LANES)]
                                     * r2_v[b, pl.ds(j * LANES, LANES)])
                    for p in perms:
                        acc = acc + jnp.take_along_axis(acc, p, axis=0)
                    vec = jnp.where(lane == l, acc, vec)
                out_v[pl.ds(off + g * LANES, LANES)] = vec
                return carry2

            lax.fori_loop(0, chunk // LANES, do_group, 0)
            return carry

        lax.fori_loop(0, n_chunks, do_chunk, 0)

        pltpu.sync_copy(out_v, out_hbm.at[pl.ds(base, b_per_w)])

    return sc_kernel


def kernel(indices, F0, F1, F2):
    B = indices.shape[0]
    V, K = F0.shape
    info = plsc.get_sparse_core_info()
    num_workers = info.num_cores * info.num_subcores
    b_per_w = B // num_workers
    chunk = min(b_per_w, 256)
    idx0 = indices[:, 0]
    idx1 = indices[:, 1]
    idx2 = indices[:, 2]
    sc = _build_sc_kernel(B, K, b_per_w, chunk, info.num_cores)
    return sc(idx0, idx1, idx2, F0, F1, F2)
